# baseline (speedup 1.0000x reference)
import jax
import jax.numpy as jnp
from jax.experimental import pallas as pl
from jax.experimental.pallas import tpu as pltpu

_RB = 8192
_LB = 16384


def _cls_body(ct_ref, cls_ref, v_ref, cp_ref, np_ref):
    t = ct_ref[...]
    posm = t != 0
    posf = posm.astype(jnp.float32)

    xt = cls_ref[...]
    m = jnp.max(xt, axis=1)
    e = jnp.exp(xt - m[:, None, :])
    s = jnp.sum(e, axis=1)
    lse = m + jnp.log(s)
    x0 = xt[:, 0, :]
    cidx = jax.lax.broadcasted_iota(jnp.int32, xt.shape, 1)
    tgt = jnp.sum(jnp.where(cidx == t[:, None, :], xt, 0.0), axis=1)

    cls_pos_part = jnp.sum((lse - tgt) * posf)
    v_ref[...] = jnp.where(posm, -jnp.inf, lse - x0)
    np_part = jnp.sum(posm.astype(jnp.int32))

    @pl.when(pl.program_id(0) == 0)
    def _():
        cp_ref[0, 0] = 0.0
        np_ref[0, 0] = 0

    cp_ref[0, 0] += cls_pos_part
    np_ref[0, 0] += np_part


def _loc_body(lp_ref, lt_ref, ct_ref, loc_ref):
    d = lp_ref[...] - lt_ref[...]
    ad = jnp.abs(d)
    sl1 = jnp.where(ad < 1.0, 0.5 * ad * ad, ad - 0.5)
    posf = (ct_ref[...] != 0).astype(jnp.float32)
    part = jnp.sum(jnp.sum(sl1, axis=1) * posf)

    @pl.when((pl.program_id(0) == 0) & (pl.program_id(1) == 0))
    def _():
        loc_ref[0, 0] = 0.0

    loc_ref[0, 0] += part


def _select_body(va_ref, vb_ref, np_ref, cp_ref, loc_ref, out_ref):
    sign = jnp.int32(-2147483648)
    va = va_ref[...]
    vb = vb_ref[...]
    m_total = 2 * va.shape[0] * va.shape[1]
    ba = jax.lax.bitcast_convert_type(va, jnp.int32)
    bb = jax.lax.bitcast_convert_type(vb, jnp.int32)
    ka = jnp.where(ba < 0, ba ^ jnp.int32(0x7FFFFFFF), ba)
    kb = jnp.where(bb < 0, bb ^ jnp.int32(0x7FFFFFFF), bb)

    npos = np_ref[0, 0]
    nneg = m_total - npos
    k = jnp.minimum(npos * 3, nneg)

    def count_ge(cand):
        cs = cand ^ sign
        return (jnp.sum((ka >= cs).astype(jnp.int32))
                + jnp.sum((kb >= cs).astype(jnp.int32)))

    def step(i, prefix):
        s = 30 - 2 * i
        c1 = prefix | jax.lax.shift_left(jnp.int32(1), s)
        c2 = prefix | jax.lax.shift_left(jnp.int32(2), s)
        c3 = prefix | jax.lax.shift_left(jnp.int32(3), s)
        n1, n2, n3 = count_ge(c1), count_ge(c2), count_ge(c3)
        return jnp.where(n3 >= k, c3,
                         jnp.where(n2 >= k, c2,
                                   jnp.where(n1 >= k, c1, prefix)))

    prefix = jax.lax.fori_loop(0, 16, step, jnp.int32(0))
    kth_skey = prefix ^ sign
    kth_bits = jnp.where(kth_skey < 0, kth_skey ^ jnp.int32(0x7FFFFFFF), kth_skey)
    kth_v = jax.lax.bitcast_convert_type(kth_bits, jnp.float32)

    gta = ka > kth_skey
    gtb = kb > kth_skey
    cnt_gt = (jnp.sum(gta.astype(jnp.int32)) + jnp.sum(gtb.astype(jnp.int32)))
    sum_gt = (jnp.sum(jnp.where(gta, va, 0.0)) + jnp.sum(jnp.where(gtb, vb, 0.0)))
    neg_sum = sum_gt + (k - cnt_gt).astype(jnp.float32) * kth_v

    npos_f = npos.astype(jnp.float32)
    out_ref[0, 0] = (cp_ref[0, 0] + neg_sum + loc_ref[0, 0]) / npos_f


def _cls_call(ct_half, clst_half, N, C):
    return pl.pallas_call(
        _cls_body,
        grid=(N // _RB,),
        in_specs=[
            pl.BlockSpec((8, _RB), lambda j: (0, j)),
            pl.BlockSpec((8, C, _RB), lambda j: (0, 0, j)),
        ],
        out_specs=[
            pl.BlockSpec((8, _RB), lambda j: (0, j)),
            pl.BlockSpec(memory_space=pltpu.SMEM),
            pl.BlockSpec(memory_space=pltpu.SMEM),
        ],
        out_shape=[
            jax.ShapeDtypeStruct((8, N), jnp.float32),
            jax.ShapeDtypeStruct((1, 1), jnp.float32),
            jax.ShapeDtypeStruct((1, 1), jnp.int32),
        ],
    )(ct_half, clst_half)


def kernel(loc_p, cls_p, loc_t, cls_t):
    B, N, C = cls_p.shape
    H = B // 2

    lpt = jnp.transpose(loc_p, (0, 2, 1))
    ltt = jnp.transpose(loc_t, (0, 2, 1))
    clst = jnp.transpose(cls_p, (0, 2, 1))
    clst_a = clst[:H]
    clst_b = clst[H:]

    smem11 = pl.BlockSpec(memory_space=pltpu.SMEM)
    va, cp_a, np_a = _cls_call(cls_t[:H], clst_a, N, C)
    vb, cp_b, np_b = _cls_call(cls_t[H:], clst_b, N, C)

    loc_s = pl.pallas_call(
        _loc_body,
        grid=(B // 8, N // _LB),
        in_specs=[
            pl.BlockSpec((8, 4, _LB), lambda b, j: (b, 0, j)),
            pl.BlockSpec((8, 4, _LB), lambda b, j: (b, 0, j)),
            pl.BlockSpec((8, _LB), lambda b, j: (b, j)),
        ],
        out_specs=pl.BlockSpec(memory_space=pltpu.SMEM),
        out_shape=jax.ShapeDtypeStruct((1, 1), jnp.float32),
    )(lpt, ltt, cls_t)

    cp_s = cp_a + cp_b
    np_i = np_a + np_b
    out = pl.pallas_call(
        _select_body,
        in_specs=[
            pl.BlockSpec(memory_space=pltpu.VMEM),
            pl.BlockSpec(memory_space=pltpu.VMEM),
            smem11, smem11, smem11,
        ],
        out_specs=pl.BlockSpec(memory_space=pltpu.SMEM),
        out_shape=jax.ShapeDtypeStruct((1, 1), jnp.float32),
    )(va, vb, np_i, cp_s, loc_s)
    return out[0, 0]


# speedup vs baseline: 1.3476x; 1.3476x over previous
import jax
import jax.numpy as jnp
from jax.experimental import pallas as pl
from jax.experimental.pallas import tpu as pltpu

_RB = 8192
_LB = 16384


def _cls_body(ct_ref, cls_ref, v_ref, cp_ref, np_ref):
    t = ct_ref[...]
    posm = t != 0
    posf = posm.astype(jnp.float32)

    xt = cls_ref[...]
    m = jnp.max(xt, axis=1)
    e = jnp.exp(xt - m[:, None, :])
    s = jnp.sum(e, axis=1)
    lse = m + jnp.log(s)
    x0 = xt[:, 0, :]
    cidx = jax.lax.broadcasted_iota(jnp.int32, xt.shape, 1)
    tgt = jnp.sum(jnp.where(cidx == t[:, None, :], xt, 0.0), axis=1)

    cls_pos_part = jnp.sum((lse - tgt) * posf)
    v_ref[...] = jnp.where(posm, -jnp.inf, lse - x0)
    np_part = jnp.sum(posm.astype(jnp.int32))

    @pl.when((pl.program_id(0) == 0) & (pl.program_id(1) == 0))
    def _():
        cp_ref[0, 0] = 0.0
        np_ref[0, 0] = 0

    cp_ref[0, 0] += cls_pos_part
    np_ref[0, 0] += np_part


def _loc_body(lp_ref, lt_ref, ct_ref, loc_ref):
    d = lp_ref[...] - lt_ref[...]
    ad = jnp.abs(d)
    sl1 = jnp.where(ad < 1.0, 0.5 * ad * ad, ad - 0.5)
    posf = (ct_ref[...] != 0).astype(jnp.float32)
    part = jnp.sum(jnp.sum(sl1, axis=1) * posf)

    @pl.when((pl.program_id(0) == 0) & (pl.program_id(1) == 0))
    def _():
        loc_ref[0, 0] = 0.0

    loc_ref[0, 0] += part


def _select_body(v_ref, np_ref, cp_ref, loc_ref, out_ref):
    sign = jnp.int32(-2147483648)
    v = v_ref[...]
    m_total = v.shape[0] * v.shape[1]
    bits = jax.lax.bitcast_convert_type(v, jnp.int32)
    skey = jnp.where(bits < 0, bits ^ jnp.int32(0x7FFFFFFF), bits)

    npos = np_ref[0, 0]
    nneg = m_total - npos
    k = jnp.minimum(npos * 3, nneg)

    def step(i, prefix):
        s = 30 - 2 * i
        c1 = prefix | jax.lax.shift_left(jnp.int32(1), s)
        c2 = prefix | jax.lax.shift_left(jnp.int32(2), s)
        c3 = prefix | jax.lax.shift_left(jnp.int32(3), s)
        n1 = jnp.sum((skey >= (c1 ^ sign)).astype(jnp.int32))
        n2 = jnp.sum((skey >= (c2 ^ sign)).astype(jnp.int32))
        n3 = jnp.sum((skey >= (c3 ^ sign)).astype(jnp.int32))
        return jnp.where(n3 >= k, c3,
                         jnp.where(n2 >= k, c2,
                                   jnp.where(n1 >= k, c1, prefix)))

    prefix = jax.lax.fori_loop(0, 16, step, jnp.int32(0))
    kth_skey = prefix ^ sign
    kth_bits = jnp.where(kth_skey < 0, kth_skey ^ jnp.int32(0x7FFFFFFF), kth_skey)
    kth_v = jax.lax.bitcast_convert_type(kth_bits, jnp.float32)

    gt = skey > kth_skey
    cnt_gt = jnp.sum(gt.astype(jnp.int32))
    sum_gt = jnp.sum(jnp.where(gt, v, 0.0))
    neg_sum = sum_gt + (k - cnt_gt).astype(jnp.float32) * kth_v

    npos_f = npos.astype(jnp.float32)
    out_ref[0, 0] = (cp_ref[0, 0] + neg_sum + loc_ref[0, 0]) / npos_f


def kernel(loc_p, cls_p, loc_t, cls_t):
    B, N, C = cls_p.shape

    lpt = jnp.transpose(loc_p, (0, 2, 1))
    ltt = jnp.transpose(loc_t, (0, 2, 1))
    clst = jnp.transpose(cls_p, (0, 2, 1))

    smem11 = pl.BlockSpec(memory_space=pltpu.SMEM)
    v, cp_s, np_i = pl.pallas_call(
        _cls_body,
        grid=(B // 8, N // _RB),
        in_specs=[
            pl.BlockSpec((8, _RB), lambda b, j: (b, j)),
            pl.BlockSpec((8, C, _RB), lambda b, j: (b, 0, j)),
        ],
        out_specs=[
            pl.BlockSpec((8, _RB), lambda b, j: (b, j)),
            pl.BlockSpec(memory_space=pltpu.SMEM),
            pl.BlockSpec(memory_space=pltpu.SMEM),
        ],
        out_shape=[
            jax.ShapeDtypeStruct((B, N), jnp.float32),
            jax.ShapeDtypeStruct((1, 1), jnp.float32),
            jax.ShapeDtypeStruct((1, 1), jnp.int32),
        ],
    )(cls_t, clst)

    loc_s = pl.pallas_call(
        _loc_body,
        grid=(B // 8, N // _LB),
        in_specs=[
            pl.BlockSpec((8, 4, _LB), lambda b, j: (b, 0, j)),
            pl.BlockSpec((8, 4, _LB), lambda b, j: (b, 0, j)),
            pl.BlockSpec((8, _LB), lambda b, j: (b, j)),
        ],
        out_specs=pl.BlockSpec(memory_space=pltpu.SMEM),
        out_shape=jax.ShapeDtypeStruct((1, 1), jnp.float32),
    )(lpt, ltt, cls_t)

    out = pl.pallas_call(
        _select_body,
        in_specs=[pl.BlockSpec(memory_space=pltpu.VMEM), smem11, smem11, smem11],
        out_specs=pl.BlockSpec(memory_space=pltpu.SMEM),
        out_shape=jax.ShapeDtypeStruct((1, 1), jnp.float32),
    )(v, np_i, cp_s, loc_s)
    return out[0, 0]
